# baseline (speedup 1.0000x reference)
import math

import jax
import jax.numpy as jnp
from jax.experimental import pallas as pl
from jax.experimental.pallas import tpu as pltpu

N = 64
IN_DIM = 128
P = 256
H = 128
HEADS = 4
DH = H // HEADS
K = 32
LAYERS = 2

_INTERPRET = False


def _dotf(a, b):
    return jnp.dot(a, b, preferred_element_type=jnp.float32)


def _dot_nt(a, b):
    return jax.lax.dot_general(a, b, (((1,), (1,)), ((), ())),
                               preferred_element_type=jnp.float32)


def _pairwise_d2(posc, posr):
    d2 = jnp.zeros((P, P), jnp.float32)
    for c in range(3):
        diff = posc[:, c:c + 1] - posr[c:c + 1, :]
        d2 = d2 + diff * diff
    ri = jax.lax.broadcasted_iota(jnp.int32, (P, P), 0)
    ci = jax.lax.broadcasted_iota(jnp.int32, (P, P), 1)
    return jnp.where(ri == ci, 1e10, d2)


def _knn_mask(d2):
    work = d2
    for _ in range(K):
        m = jnp.min(work, axis=1, keepdims=True)
        work = jnp.where(work <= m, jnp.float32(3e38), work)
    return jnp.where(work >= 1e38, 1.0, 0.0)


def _embed_kernel(x_ref, w_ref, b_ref, o_ref):
    o_ref[...] = _dotf(x_ref[...], w_ref[...]) + b_ref[...]


def _knn0_kernel(posc_ref, posr_ref, eaT_ref, oh_ref, eac_ref, distc_ref):
    d2 = _pairwise_d2(posc_ref[...], posr_ref[...])
    eaT = eaT_ref[...]
    work = d2
    ea_cols = []
    dist_cols = []
    for t in range(K):
        m = jnp.min(work, axis=1, keepdims=True)
        sel = work <= m
        self_f = jnp.where(sel, 1.0, 0.0)
        oh_ref[t * P:(t + 1) * P, :] = self_f
        ea_cols.append(jnp.sum(self_f * eaT, axis=1, keepdims=True))
        dist_cols.append(jnp.sqrt(m))
        work = jnp.where(sel, jnp.float32(3e38), work)
    eac_ref[...] = jnp.concatenate(ea_cols, axis=1)
    distc_ref[...] = jnp.concatenate(dist_cols, axis=1)


def _graph_kernel(h0_ref, posc_ref, eaT_ref, oh0_ref, eac_ref, distc_ref,
                  wq_ref, bq_ref, wk_ref, bk_ref, wv_ref, bv_ref,
                  wo_ref, bo_ref, lin_w_ref, lin_b_ref,
                  w1a_ref, w1b_ref, w_ea_ref, w_d_ref, e_b1_ref,
                  e_w2_ref, e_w2c_ref, e_b2_ref,
                  n_w1h_ref, n_w1e_ref, n_b1_ref, n_w2_ref, n_b2_ref,
                  c_w1_ref, c_b1_ref, c_w2_ref, c_b2_ref,
                  out1_ref, out2_ref):
    h = h0_ref[0]

    q = _dotf(h, wq_ref[...]) + bq_ref[...]
    k = _dotf(h, wk_ref[...]) + bk_ref[...]
    v = _dotf(h, wv_ref[...]) + bv_ref[...]
    scale = 1.0 / math.sqrt(DH)
    heads = []
    for hd in range(HEADS):
        sl = slice(hd * DH, (hd + 1) * DH)
        s = _dot_nt(q[:, sl], k[:, sl]) * scale
        s = s - jnp.max(s, axis=1, keepdims=True)
        e = jnp.exp(s)
        pattn = e / jnp.sum(e, axis=1, keepdims=True)
        heads.append(_dotf(pattn, v[:, sl]))
    o = jnp.concatenate(heads, axis=1)
    o = _dotf(o, wo_ref[...]) + bo_ref[...]
    h = _dotf(o, lin_w_ref[...]) + lin_b_ref[...]

    a = _dotf(h, w1a_ref[0]) + e_b1_ref[0]
    b = _dotf(h, w1b_ref[0])
    Bg = _dotf(oh0_ref[...], b)
    w_ea = w_ea_ref[0]
    w_d = w_d_ref[0]
    eac = eac_ref[...]
    distc = distc_ref[...]
    r_acc = jnp.zeros((P, 16), jnp.float32)
    for t in range(K):
        u = (a + Bg[t * P:(t + 1) * P, :]
             + eac[:, t:t + 1] * w_ea
             + distc[:, t:t + 1] * w_d)
        r_acc = r_acc + jnp.maximum(u, 0.0)
    e_mean = (_dotf(r_acc, e_w2c_ref[0]) * (1.0 / K)) + e_b2_ref[0]

    pre = _dotf(h, n_w1h_ref[0]) + e_mean * n_w1e_ref[0] + n_b1_ref[0]
    h = _dotf(jnp.maximum(pre, 0.0), n_w2_ref[0]) + n_b2_ref[0]

    g = jnp.maximum(e_mean * c_w1_ref[0] + c_b1_ref[0], 0.0)
    dpos = _dotf(g, c_w2_ref[0]) + c_b2_ref[0]
    posc = posc_ref[...] + dpos
    out1_ref[0] = posc

    posr = posc.T
    G = _dotf(posc, posr)
    n2c = jnp.sum(posc * posc, axis=1, keepdims=True)
    n2r = jnp.sum(posr * posr, axis=0, keepdims=True)
    d2 = jnp.maximum(n2c + n2r - 2.0 * G, 0.0)
    ri = jax.lax.broadcasted_iota(jnp.int32, (P, P), 0)
    ci = jax.lax.broadcasted_iota(jnp.int32, (P, P), 1)
    d2 = jnp.where(ri == ci, 1e10, d2)
    dist = jnp.sqrt(d2)
    mask = _knn_mask(d2)

    a = _dotf(h, w1a_ref[1]) + e_b1_ref[1]
    b = _dotf(h, w1b_ref[1])
    bT = b.T
    w_ea = w_ea_ref[1]
    w_d = w_d_ref[1]
    w2 = e_w2_ref[1]
    eaT = eaT_ref[...]
    F = jnp.zeros((P, P), jnp.float32)
    for m in range(16):
        u = (a[:, m:m + 1] + bT[m:m + 1, :]
             + eaT * w_ea[0:1, m:m + 1]
             + dist * w_d[0:1, m:m + 1])
        F = F + jnp.maximum(u, 0.0) * w2[0:1, m:m + 1]
    s = jnp.sum(mask * F, axis=1, keepdims=True)
    e_mean = s * (1.0 / K) + e_b2_ref[1]

    g = jnp.maximum(e_mean * c_w1_ref[1] + c_b1_ref[1], 0.0)
    dpos = _dotf(g, c_w2_ref[1]) + c_b2_ref[1]
    out2_ref[0] = posc + dpos


def _full(shape):
    rank = len(shape)
    return pl.BlockSpec(shape, lambda *_: (0,) * rank)


@jax.jit
def kernel(x, pos, edge_attr, params):
    f32 = jnp.float32

    CB = 4096
    n_cb = (P * H) // CB
    h0 = pl.pallas_call(
        _embed_kernel,
        grid=(n_cb,),
        in_specs=[
            pl.BlockSpec((N, IN_DIM), lambda i: (0, 0)),
            pl.BlockSpec((IN_DIM, CB), lambda i: (0, i)),
            pl.BlockSpec((1, CB), lambda i: (0, i)),
        ],
        out_specs=pl.BlockSpec((N, CB), lambda i: (0, i)),
        out_shape=jax.ShapeDtypeStruct((N, P * H), f32),
        compiler_params=pltpu.CompilerParams(
            dimension_semantics=("arbitrary",)),
        interpret=_INTERPRET,
    )(x, params['z_W'], params['z_b'].reshape(1, P * H))
    h0 = h0.reshape(N, P, H)

    posc = pos.astype(f32)
    posr = posc.T
    eaT = edge_attr.T

    oh0, eac, distc = pl.pallas_call(
        _knn0_kernel,
        in_specs=[_full((P, 3)), _full((3, P)), _full((P, P))],
        out_specs=[_full((K * P, P)), _full((P, K)), _full((P, K))],
        out_shape=[jax.ShapeDtypeStruct((K * P, P), f32),
                   jax.ShapeDtypeStruct((P, K), f32),
                   jax.ShapeDtypeStruct((P, K), f32)],
        interpret=_INTERPRET,
    )(posc, posr, eaT)

    lp = params['layers']

    def stack(name):
        return jnp.stack([lp[l][name] for l in range(LAYERS)])

    e_W1 = stack('e_W1')
    w1a = e_W1[:, :H, :]
    w1b = e_W1[:, H:2 * H, :]
    w_ea = e_W1[:, 2 * H:2 * H + 1, :]
    w_d = e_W1[:, 2 * H + 1:2 * H + 2, :]
    e_b1 = stack('e_b1').reshape(LAYERS, 1, 16)
    e_w2 = stack('e_W2').reshape(LAYERS, 1, 16)
    e_w2c = stack('e_W2')
    e_b2 = stack('e_b2').reshape(LAYERS, 1, 1)
    n_W1 = stack('n_W1')
    n_w1h = n_W1[:, :H, :]
    n_w1e = n_W1[:, H:H + 1, :]
    n_b1 = stack('n_b1').reshape(LAYERS, 1, 16)
    n_w2 = stack('n_W2')
    n_b2 = stack('n_b2').reshape(LAYERS, 1, H)
    c_w1 = stack('c_W1')
    c_b1 = stack('c_b1').reshape(LAYERS, 1, 16)
    c_w2 = stack('c_W2')
    c_b2 = stack('c_b2').reshape(LAYERS, 1, 3)

    in_specs = [
        pl.BlockSpec((1, P, H), lambda n: (n, 0, 0)),
        _full((P, 3)), _full((P, P)),
        _full((K * P, P)), _full((P, K)), _full((P, K)),
        _full((H, H)), _full((1, H)), _full((H, H)), _full((1, H)),
        _full((H, H)), _full((1, H)), _full((H, H)), _full((1, H)),
        _full((H, H)), _full((1, H)),
        _full((LAYERS, H, 16)), _full((LAYERS, H, 16)),
        _full((LAYERS, 1, 16)), _full((LAYERS, 1, 16)),
        _full((LAYERS, 1, 16)), _full((LAYERS, 1, 16)),
        _full((LAYERS, 16, 1)), _full((LAYERS, 1, 1)),
        _full((LAYERS, H, 16)), _full((LAYERS, 1, 16)),
        _full((LAYERS, 1, 16)), _full((LAYERS, 16, H)),
        _full((LAYERS, 1, H)),
        _full((LAYERS, 1, 16)), _full((LAYERS, 1, 16)),
        _full((LAYERS, 16, 3)), _full((LAYERS, 1, 3)),
    ]
    out_spec = pl.BlockSpec((1, P, 3), lambda n: (n, 0, 0))
    c1, c2 = pl.pallas_call(
        _graph_kernel,
        grid=(N,),
        in_specs=in_specs,
        out_specs=[out_spec, out_spec],
        out_shape=[jax.ShapeDtypeStruct((N, P, 3), f32),
                   jax.ShapeDtypeStruct((N, P, 3), f32)],
        compiler_params=pltpu.CompilerParams(
            dimension_semantics=("parallel",)),
        interpret=_INTERPRET,
    )(h0, posc, eaT, oh0, eac, distc,
      params['Wq'], params['bq'].reshape(1, H),
      params['Wk'], params['bk'].reshape(1, H),
      params['Wv'], params['bv'].reshape(1, H),
      params['Wo'], params['bo'].reshape(1, H),
      params['lin_W'], params['lin_b'].reshape(1, H),
      w1a, w1b, w_ea, w_d, e_b1, e_w2, e_w2c, e_b2,
      n_w1h, n_w1e, n_b1, n_w2, n_b2,
      c_w1, c_b1, c_w2, c_b2)

    return (c1.reshape(N * P, 3), c2.reshape(N * P, 3))


# speedup vs baseline: 1.0890x; 1.0890x over previous
import math

import jax
import jax.numpy as jnp
from jax.experimental import pallas as pl
from jax.experimental.pallas import tpu as pltpu

N = 64
IN_DIM = 128
P = 256
H = 128
HEADS = 4
DH = H // HEADS
K = 32
LAYERS = 2

_INTERPRET = False


def _dotf(a, b):
    return jnp.dot(a, b, preferred_element_type=jnp.float32)


def _dot_nt(a, b):
    return jax.lax.dot_general(a, b, (((1,), (1,)), ((), ())),
                               preferred_element_type=jnp.float32)


def _pairwise_d2(posc, posr):
    d2 = jnp.zeros((P, P), jnp.float32)
    for c in range(3):
        diff = posc[:, c:c + 1] - posr[c:c + 1, :]
        d2 = d2 + diff * diff
    ri = jax.lax.broadcasted_iota(jnp.int32, (P, P), 0)
    ci = jax.lax.broadcasted_iota(jnp.int32, (P, P), 1)
    return jnp.where(ri == ci, 1e10, d2)


def _knn_mask(d2):
    work = d2
    for _ in range(K):
        m = jnp.min(work, axis=1, keepdims=True)
        work = jnp.where(work <= m, jnp.float32(3e38), work)
    return jnp.where(work >= 1e38, 1.0, 0.0)


def _embed_kernel(x_ref, w_ref, b_ref, o_ref):
    o_ref[...] = _dotf(x_ref[...], w_ref[...]) + b_ref[...]


def _knn0_kernel(posc_ref, posr_ref, mask_ref, dist_ref):
    d2 = _pairwise_d2(posc_ref[...], posr_ref[...])
    dist_ref[...] = jnp.sqrt(d2)
    mask_ref[...] = _knn_mask(d2)


def _graph_kernel(h0_ref, posc_ref, eaT_ref, mask0_ref, dist0_ref,
                  wq_ref, bq_ref, wk_ref, bk_ref, wv_ref, bv_ref,
                  wo_ref, bo_ref, lin_w_ref, lin_b_ref,
                  w1a_ref, w1b_ref, w_ea_ref, w_d_ref, e_b1_ref,
                  e_w2_ref, e_b2_ref,
                  n_w1h_ref, n_w1e_ref, n_b1_ref, n_w2_ref, n_b2_ref,
                  c_w1_ref, c_b1_ref, c_w2_ref, c_b2_ref,
                  out1_ref, out2_ref):
    h = h0_ref[0]

    q = _dotf(h, wq_ref[...]) + bq_ref[...]
    k = _dotf(h, wk_ref[...]) + bk_ref[...]
    v = _dotf(h, wv_ref[...]) + bv_ref[...]
    scale = 1.0 / math.sqrt(DH)
    heads = []
    for hd in range(HEADS):
        sl = slice(hd * DH, (hd + 1) * DH)
        s = _dot_nt(q[:, sl], k[:, sl]) * scale
        s = s - jnp.max(s, axis=1, keepdims=True)
        e = jnp.exp(s)
        pattn = e / jnp.sum(e, axis=1, keepdims=True)
        heads.append(_dotf(pattn, v[:, sl]))
    o = jnp.concatenate(heads, axis=1)
    o = _dotf(o, wo_ref[...]) + bo_ref[...]
    h = _dotf(o, lin_w_ref[...]) + lin_b_ref[...]

    eaT = eaT_ref[...]
    a = _dotf(h, w1a_ref[0]) + e_b1_ref[0]
    b = _dotf(h, w1b_ref[0])
    bT = b.T
    w_ea = w_ea_ref[0]
    w_d = w_d_ref[0]
    w2 = e_w2_ref[0]
    dist0 = dist0_ref[...]
    F = jnp.zeros((P, P), jnp.float32)
    for m in range(16):
        u = (a[:, m:m + 1] + bT[m:m + 1, :]
             + eaT * w_ea[0:1, m:m + 1]
             + dist0 * w_d[0:1, m:m + 1])
        F = F + jnp.maximum(u, 0.0) * w2[0:1, m:m + 1]
    s = jnp.sum(mask0_ref[...] * F, axis=1, keepdims=True)
    e_mean = s * (1.0 / K) + e_b2_ref[0]

    pre = _dotf(h, n_w1h_ref[0]) + e_mean * n_w1e_ref[0] + n_b1_ref[0]
    h = _dotf(jnp.maximum(pre, 0.0), n_w2_ref[0]) + n_b2_ref[0]

    g = jnp.maximum(e_mean * c_w1_ref[0] + c_b1_ref[0], 0.0)
    dpos = _dotf(g, c_w2_ref[0]) + c_b2_ref[0]
    posc = posc_ref[...] + dpos
    out1_ref[0] = posc

    posr = posc.T
    G = _dotf(posc, posr)
    n2c = jnp.sum(posc * posc, axis=1, keepdims=True)
    n2r = jnp.sum(posr * posr, axis=0, keepdims=True)
    d2 = jnp.maximum(n2c + n2r - 2.0 * G, 0.0)
    ri = jax.lax.broadcasted_iota(jnp.int32, (P, P), 0)
    ci = jax.lax.broadcasted_iota(jnp.int32, (P, P), 1)
    d2 = jnp.where(ri == ci, 1e10, d2)
    dist = jnp.sqrt(d2)
    mask = _knn_mask(d2)

    a = _dotf(h, w1a_ref[1]) + e_b1_ref[1]
    b = _dotf(h, w1b_ref[1])
    bT = b.T
    w_ea = w_ea_ref[1]
    w_d = w_d_ref[1]
    w2 = e_w2_ref[1]
    F = jnp.zeros((P, P), jnp.float32)
    for m in range(16):
        u = (a[:, m:m + 1] + bT[m:m + 1, :]
             + eaT * w_ea[0:1, m:m + 1]
             + dist * w_d[0:1, m:m + 1])
        F = F + jnp.maximum(u, 0.0) * w2[0:1, m:m + 1]
    s = jnp.sum(mask * F, axis=1, keepdims=True)
    e_mean = s * (1.0 / K) + e_b2_ref[1]

    g = jnp.maximum(e_mean * c_w1_ref[1] + c_b1_ref[1], 0.0)
    dpos = _dotf(g, c_w2_ref[1]) + c_b2_ref[1]
    out2_ref[0] = posc + dpos


def _full(shape):
    rank = len(shape)
    return pl.BlockSpec(shape, lambda *_: (0,) * rank)


@jax.jit
def kernel(x, pos, edge_attr, params):
    f32 = jnp.float32

    CB = 4096
    n_cb = (P * H) // CB
    h0 = pl.pallas_call(
        _embed_kernel,
        grid=(n_cb,),
        in_specs=[
            pl.BlockSpec((N, IN_DIM), lambda i: (0, 0)),
            pl.BlockSpec((IN_DIM, CB), lambda i: (0, i)),
            pl.BlockSpec((1, CB), lambda i: (0, i)),
        ],
        out_specs=pl.BlockSpec((N, CB), lambda i: (0, i)),
        out_shape=jax.ShapeDtypeStruct((N, P * H), f32),
        compiler_params=pltpu.CompilerParams(
            dimension_semantics=("arbitrary",)),
        interpret=_INTERPRET,
    )(x, params['z_W'], params['z_b'].reshape(1, P * H))
    h0 = h0.reshape(N, P, H)

    posc = pos.astype(f32)
    posr = posc.T
    eaT = edge_attr.T

    mask0, dist0 = pl.pallas_call(
        _knn0_kernel,
        in_specs=[_full((P, 3)), _full((3, P))],
        out_specs=[_full((P, P)), _full((P, P))],
        out_shape=[jax.ShapeDtypeStruct((P, P), f32),
                   jax.ShapeDtypeStruct((P, P), f32)],
        interpret=_INTERPRET,
    )(posc, posr)

    lp = params['layers']

    def stack(name):
        return jnp.stack([lp[l][name] for l in range(LAYERS)])

    e_W1 = stack('e_W1')
    w1a = e_W1[:, :H, :]
    w1b = e_W1[:, H:2 * H, :]
    w_ea = e_W1[:, 2 * H:2 * H + 1, :]
    w_d = e_W1[:, 2 * H + 1:2 * H + 2, :]
    e_b1 = stack('e_b1').reshape(LAYERS, 1, 16)
    e_w2 = stack('e_W2').reshape(LAYERS, 1, 16)
    e_b2 = stack('e_b2').reshape(LAYERS, 1, 1)
    n_W1 = stack('n_W1')
    n_w1h = n_W1[:, :H, :]
    n_w1e = n_W1[:, H:H + 1, :]
    n_b1 = stack('n_b1').reshape(LAYERS, 1, 16)
    n_w2 = stack('n_W2')
    n_b2 = stack('n_b2').reshape(LAYERS, 1, H)
    c_w1 = stack('c_W1')
    c_b1 = stack('c_b1').reshape(LAYERS, 1, 16)
    c_w2 = stack('c_W2')
    c_b2 = stack('c_b2').reshape(LAYERS, 1, 3)

    in_specs = [
        pl.BlockSpec((1, P, H), lambda n: (n, 0, 0)),
        _full((P, 3)), _full((P, P)),
        _full((P, P)), _full((P, P)),
        _full((H, H)), _full((1, H)), _full((H, H)), _full((1, H)),
        _full((H, H)), _full((1, H)), _full((H, H)), _full((1, H)),
        _full((H, H)), _full((1, H)),
        _full((LAYERS, H, 16)), _full((LAYERS, H, 16)),
        _full((LAYERS, 1, 16)), _full((LAYERS, 1, 16)),
        _full((LAYERS, 1, 16)), _full((LAYERS, 1, 16)),
        _full((LAYERS, 1, 1)),
        _full((LAYERS, H, 16)), _full((LAYERS, 1, 16)),
        _full((LAYERS, 1, 16)), _full((LAYERS, 16, H)),
        _full((LAYERS, 1, H)),
        _full((LAYERS, 1, 16)), _full((LAYERS, 1, 16)),
        _full((LAYERS, 16, 3)), _full((LAYERS, 1, 3)),
    ]
    out_spec = pl.BlockSpec((1, P, 3), lambda n: (n, 0, 0))
    c1, c2 = pl.pallas_call(
        _graph_kernel,
        grid=(N,),
        in_specs=in_specs,
        out_specs=[out_spec, out_spec],
        out_shape=[jax.ShapeDtypeStruct((N, P, 3), f32),
                   jax.ShapeDtypeStruct((N, P, 3), f32)],
        compiler_params=pltpu.CompilerParams(
            dimension_semantics=("parallel",)),
        interpret=_INTERPRET,
    )(h0, posc, eaT, mask0, dist0,
      params['Wq'], params['bq'].reshape(1, H),
      params['Wk'], params['bk'].reshape(1, H),
      params['Wv'], params['bv'].reshape(1, H),
      params['Wo'], params['bo'].reshape(1, H),
      params['lin_W'], params['lin_b'].reshape(1, H),
      w1a, w1b, w_ea, w_d, e_b1, e_w2, e_b2,
      n_w1h, n_w1e, n_b1, n_w2, n_b2,
      c_w1, c_b1, c_w2, c_b2)

    return (c1.reshape(N * P, 3), c2.reshape(N * P, 3))


# speedup vs baseline: 1.4226x; 1.3064x over previous
import math

import jax
import jax.numpy as jnp
from jax.experimental import pallas as pl
from jax.experimental.pallas import tpu as pltpu

N = 64
IN_DIM = 128
P = 256
H = 128
HEADS = 4
DH = H // HEADS
K = 32
LAYERS = 2

_INTERPRET = False


def _dotf(a, b):
    return jnp.dot(a, b, preferred_element_type=jnp.float32)


def _dot_nt(a, b):
    return jax.lax.dot_general(a, b, (((1,), (1,)), ((), ())),
                               preferred_element_type=jnp.float32)


def _pairwise_d2(posc, posr):
    d2 = jnp.zeros((P, P), jnp.float32)
    for c in range(3):
        diff = posc[:, c:c + 1] - posr[c:c + 1, :]
        d2 = d2 + diff * diff
    ri = jax.lax.broadcasted_iota(jnp.int32, (P, P), 0)
    ci = jax.lax.broadcasted_iota(jnp.int32, (P, P), 1)
    return jnp.where(ri == ci, 1e10, d2)


def _knn_mask(d2):
    work = d2
    for _ in range(K):
        m = jnp.min(work, axis=1, keepdims=True)
        work = jnp.where(work <= m, jnp.float32(3e38), work)
    return jnp.where(work >= 1e38, 1.0, 0.0)


def _embed_kernel(x_ref, w_ref, b_ref, o_ref):
    o_ref[...] = _dotf(x_ref[...], w_ref[...]) + b_ref[...]


def _knn0_kernel(posc_ref, posr_ref, mask_ref, dist_ref):
    d2 = _pairwise_d2(posc_ref[...], posr_ref[...])
    dist_ref[...] = jnp.sqrt(d2)
    mask_ref[...] = _knn_mask(d2)


def _graph_kernel(h0_ref, posc_ref, eaT_ref, mask0_ref, dist0_ref,
                  wq_ref, bq_ref, wk_ref, bk_ref, wv_ref, bv_ref,
                  wo_ref, bo_ref, lin_w_ref, lin_b_ref,
                  w1a_ref, w1b_ref, w_ea_ref, w_d_ref, e_b1_ref,
                  e_w2_ref, e_b2_ref,
                  n_w1h_ref, n_w1e_ref, n_b1_ref, n_w2_ref, n_b2_ref,
                  c_w1_ref, c_b1_ref, c_w2_ref, c_b2_ref,
                  out1_ref, out2_ref):
    h = h0_ref[0]

    q = _dotf(h, wq_ref[...]) + bq_ref[...]
    k = _dotf(h, wk_ref[...]) + bk_ref[...]
    v = _dotf(h, wv_ref[...]) + bv_ref[...]
    scale = 1.0 / math.sqrt(DH)
    heads = []
    for hd in range(HEADS):
        sl = slice(hd * DH, (hd + 1) * DH)
        s = _dot_nt(q[:, sl], k[:, sl]) * scale
        s = s - jnp.max(s, axis=1, keepdims=True)
        e = jnp.exp(s)
        pattn = e / jnp.sum(e, axis=1, keepdims=True)
        heads.append(_dotf(pattn, v[:, sl]))
    o = jnp.concatenate(heads, axis=1)
    o = _dotf(o, wo_ref[...]) + bo_ref[...]
    h = _dotf(o, lin_w_ref[...]) + lin_b_ref[...]

    eaT = eaT_ref[...]
    a = _dotf(h, w1a_ref[0]) + e_b1_ref[0]
    b = _dotf(h, w1b_ref[0])
    bT = b.T
    w_ea = w_ea_ref[0]
    w_d = w_d_ref[0]
    w2 = e_w2_ref[0]
    dist0 = dist0_ref[...]
    F = jnp.zeros((P, P), jnp.float32)
    for m in range(16):
        u = (a[:, m:m + 1] + bT[m:m + 1, :]
             + eaT * w_ea[0:1, m:m + 1]
             + dist0 * w_d[0:1, m:m + 1])
        F = F + jnp.maximum(u, 0.0) * w2[0:1, m:m + 1]
    s = jnp.sum(mask0_ref[...] * F, axis=1, keepdims=True)
    e_mean = s * (1.0 / K) + e_b2_ref[0]

    pre = _dotf(h, n_w1h_ref[0]) + e_mean * n_w1e_ref[0] + n_b1_ref[0]
    h = _dotf(jnp.maximum(pre, 0.0), n_w2_ref[0]) + n_b2_ref[0]

    g = jnp.maximum(e_mean * c_w1_ref[0] + c_b1_ref[0], 0.0)
    dpos = _dotf(g, c_w2_ref[0]) + c_b2_ref[0]
    posc = posc_ref[...] + dpos
    out1_ref[0] = posc

    posr = posc.T
    d2 = _pairwise_d2(posc, posr)
    dist = jnp.sqrt(d2)
    mask = _knn_mask(d2)

    a = _dotf(h, w1a_ref[1]) + e_b1_ref[1]
    b = _dotf(h, w1b_ref[1])
    bT = b.T
    w_ea = w_ea_ref[1]
    w_d = w_d_ref[1]
    w2 = e_w2_ref[1]
    F = jnp.zeros((P, P), jnp.float32)
    for m in range(16):
        u = (a[:, m:m + 1] + bT[m:m + 1, :]
             + eaT * w_ea[0:1, m:m + 1]
             + dist * w_d[0:1, m:m + 1])
        F = F + jnp.maximum(u, 0.0) * w2[0:1, m:m + 1]
    s = jnp.sum(mask * F, axis=1, keepdims=True)
    e_mean = s * (1.0 / K) + e_b2_ref[1]

    g = jnp.maximum(e_mean * c_w1_ref[1] + c_b1_ref[1], 0.0)
    dpos = _dotf(g, c_w2_ref[1]) + c_b2_ref[1]
    out2_ref[0] = posc + dpos


def _full(shape):
    rank = len(shape)
    return pl.BlockSpec(shape, lambda *_: (0,) * rank)


@jax.jit
def kernel(x, pos, edge_attr, params):
    f32 = jnp.float32

    CB = 4096
    n_cb = (P * H) // CB
    h0 = pl.pallas_call(
        _embed_kernel,
        grid=(n_cb,),
        in_specs=[
            pl.BlockSpec((N, IN_DIM), lambda i: (0, 0)),
            pl.BlockSpec((IN_DIM, CB), lambda i: (0, i)),
            pl.BlockSpec((1, CB), lambda i: (0, i)),
        ],
        out_specs=pl.BlockSpec((N, CB), lambda i: (0, i)),
        out_shape=jax.ShapeDtypeStruct((N, P * H), f32),
        compiler_params=pltpu.CompilerParams(
            dimension_semantics=("arbitrary",)),
        interpret=_INTERPRET,
    )(x, params['z_W'], params['z_b'].reshape(1, P * H))
    h0 = h0.reshape(N, P, H)

    posc = pos.astype(f32)
    posr = posc.T
    eaT = edge_attr.T

    mask0, dist0 = pl.pallas_call(
        _knn0_kernel,
        in_specs=[_full((P, 3)), _full((3, P))],
        out_specs=[_full((P, P)), _full((P, P))],
        out_shape=[jax.ShapeDtypeStruct((P, P), f32),
                   jax.ShapeDtypeStruct((P, P), f32)],
        interpret=_INTERPRET,
    )(posc, posr)

    lp = params['layers']

    def stack(name):
        return jnp.stack([lp[l][name] for l in range(LAYERS)])

    e_W1 = stack('e_W1')
    w1a = e_W1[:, :H, :]
    w1b = e_W1[:, H:2 * H, :]
    w_ea = e_W1[:, 2 * H:2 * H + 1, :]
    w_d = e_W1[:, 2 * H + 1:2 * H + 2, :]
    e_b1 = stack('e_b1').reshape(LAYERS, 1, 16)
    e_w2 = stack('e_W2').reshape(LAYERS, 1, 16)
    e_b2 = stack('e_b2').reshape(LAYERS, 1, 1)
    n_W1 = stack('n_W1')
    n_w1h = n_W1[:, :H, :]
    n_w1e = n_W1[:, H:H + 1, :]
    n_b1 = stack('n_b1').reshape(LAYERS, 1, 16)
    n_w2 = stack('n_W2')
    n_b2 = stack('n_b2').reshape(LAYERS, 1, H)
    c_w1 = stack('c_W1')
    c_b1 = stack('c_b1').reshape(LAYERS, 1, 16)
    c_w2 = stack('c_W2')
    c_b2 = stack('c_b2').reshape(LAYERS, 1, 3)

    in_specs = [
        pl.BlockSpec((1, P, H), lambda n: (n, 0, 0)),
        _full((P, 3)), _full((P, P)),
        _full((P, P)), _full((P, P)),
        _full((H, H)), _full((1, H)), _full((H, H)), _full((1, H)),
        _full((H, H)), _full((1, H)), _full((H, H)), _full((1, H)),
        _full((H, H)), _full((1, H)),
        _full((LAYERS, H, 16)), _full((LAYERS, H, 16)),
        _full((LAYERS, 1, 16)), _full((LAYERS, 1, 16)),
        _full((LAYERS, 1, 16)), _full((LAYERS, 1, 16)),
        _full((LAYERS, 1, 1)),
        _full((LAYERS, H, 16)), _full((LAYERS, 1, 16)),
        _full((LAYERS, 1, 16)), _full((LAYERS, 16, H)),
        _full((LAYERS, 1, H)),
        _full((LAYERS, 1, 16)), _full((LAYERS, 1, 16)),
        _full((LAYERS, 16, 3)), _full((LAYERS, 1, 3)),
    ]
    out_spec = pl.BlockSpec((1, P, 3), lambda n: (n, 0, 0))
    c1, c2 = pl.pallas_call(
        _graph_kernel,
        grid=(N,),
        in_specs=in_specs,
        out_specs=[out_spec, out_spec],
        out_shape=[jax.ShapeDtypeStruct((N, P, 3), f32),
                   jax.ShapeDtypeStruct((N, P, 3), f32)],
        compiler_params=pltpu.CompilerParams(
            dimension_semantics=("parallel",)),
        interpret=_INTERPRET,
    )(h0, posc, eaT, mask0, dist0,
      params['Wq'], params['bq'].reshape(1, H),
      params['Wk'], params['bk'].reshape(1, H),
      params['Wv'], params['bv'].reshape(1, H),
      params['Wo'], params['bo'].reshape(1, H),
      params['lin_W'], params['lin_b'].reshape(1, H),
      w1a, w1b, w_ea, w_d, e_b1, e_w2, e_b2,
      n_w1h, n_w1e, n_b1, n_w2, n_b2,
      c_w1, c_b1, c_w2, c_b2)

    return (c1.reshape(N * P, 3), c2.reshape(N * P, 3))


# speedup vs baseline: 1.8597x; 1.3073x over previous
import math

import jax
import jax.numpy as jnp
from jax.experimental import pallas as pl
from jax.experimental.pallas import tpu as pltpu

N = 64
IN_DIM = 128
P = 256
H = 128
HEADS = 4
DH = H // HEADS
K = 32
LAYERS = 2

_INTERPRET = False


def _dotf(a, b):
    return jnp.dot(a, b, preferred_element_type=jnp.float32)


def _dot_nt(a, b):
    return jax.lax.dot_general(a, b, (((1,), (1,)), ((), ())),
                               preferred_element_type=jnp.float32)


def _pairwise_d2(posc, posr):
    d2 = jnp.zeros((P, P), jnp.float32)
    for c in range(3):
        diff = posc[:, c:c + 1] - posr[c:c + 1, :]
        d2 = d2 + diff * diff
    ri = jax.lax.broadcasted_iota(jnp.int32, (P, P), 0)
    ci = jax.lax.broadcasted_iota(jnp.int32, (P, P), 1)
    return jnp.where(ri == ci, 1e10, d2)


def _knn_mask(d2):
    work = d2
    for _ in range(K):
        m = jnp.min(work, axis=1, keepdims=True)
        work = jnp.where(work <= m, jnp.float32(3e38), work)
    return jnp.where(work >= 1e38, 1.0, 0.0)


def _embed_kernel(x_ref, w_ref, b_ref, o_ref):
    o_ref[...] = _dotf(x_ref[...], w_ref[...]) + b_ref[...]


def _knn0_kernel(posc_ref, posr_ref, eaT_ref, w_ea_ref, w_d_ref,
                 mask_ref, c0_ref, ea1_ref):
    d2 = _pairwise_d2(posc_ref[...], posr_ref[...])
    dist0 = jnp.sqrt(d2)
    mask_ref[...] = _knn_mask(d2)
    eaT = eaT_ref[...]
    for m in range(16):
        c0 = (eaT * w_ea_ref[0, 0:1, m:m + 1]
              + dist0 * w_d_ref[0, 0:1, m:m + 1])
        c0_ref[m] = c0.astype(jnp.bfloat16)
        ea1_ref[m] = (eaT * w_ea_ref[1, 0:1, m:m + 1]).astype(jnp.bfloat16)


def _graph_kernel(h0_ref, posc_ref, mask0_ref, c0_ref, ea1_ref,
                  wq_ref, bq_ref, wk_ref, bk_ref, wv_ref, bv_ref,
                  wo_ref, bo_ref, lin_w_ref, lin_b_ref,
                  w1a_ref, w1b_ref, w_ea_ref, w_d_ref, e_b1_ref,
                  e_w2_ref, e_b2_ref,
                  n_w1h_ref, n_w1e_ref, n_b1_ref, n_w2_ref, n_b2_ref,
                  c_w1_ref, c_b1_ref, c_w2_ref, c_b2_ref,
                  out1_ref, out2_ref):
    h = h0_ref[0]

    q = _dotf(h, wq_ref[...]) + bq_ref[...]
    k = _dotf(h, wk_ref[...]) + bk_ref[...]
    v = _dotf(h, wv_ref[...]) + bv_ref[...]
    scale = 1.0 / math.sqrt(DH)
    heads = []
    for hd in range(HEADS):
        sl = slice(hd * DH, (hd + 1) * DH)
        s = _dot_nt(q[:, sl], k[:, sl]) * scale
        e = jnp.exp(s)
        pattn = e / jnp.sum(e, axis=1, keepdims=True)
        heads.append(_dotf(pattn, v[:, sl]))
    o = jnp.concatenate(heads, axis=1)
    o = _dotf(o, wo_ref[...]) + bo_ref[...]
    h = _dotf(o, lin_w_ref[...]) + lin_b_ref[...]

    bf16 = jnp.bfloat16
    a = _dotf(h, w1a_ref[0]) + e_b1_ref[0]
    b = _dotf(h, w1b_ref[0])
    a16 = a.astype(bf16)
    bT16 = b.T.astype(bf16)
    w2 = e_w2_ref[0].astype(bf16)
    F = jnp.zeros((P, P), bf16)
    for m in range(16):
        u = a16[:, m:m + 1] + bT16[m:m + 1, :] + c0_ref[m]
        F = F + jnp.maximum(u, jnp.bfloat16(0.0)) * w2[0:1, m:m + 1]
    s = jnp.sum(mask0_ref[...] * F.astype(jnp.float32), axis=1,
                keepdims=True)
    e_mean = s * (1.0 / K) + e_b2_ref[0]

    pre = _dotf(h, n_w1h_ref[0]) + e_mean * n_w1e_ref[0] + n_b1_ref[0]
    h = _dotf(jnp.maximum(pre, 0.0), n_w2_ref[0]) + n_b2_ref[0]

    g = jnp.maximum(e_mean * c_w1_ref[0] + c_b1_ref[0], 0.0)
    dpos = _dotf(g, c_w2_ref[0]) + c_b2_ref[0]
    posc = posc_ref[...] + dpos
    out1_ref[0] = posc

    posr = posc.T
    d2 = _pairwise_d2(posc, posr)
    dist = jnp.sqrt(d2)
    mask = _knn_mask(d2)

    a = _dotf(h, w1a_ref[1]) + e_b1_ref[1]
    b = _dotf(h, w1b_ref[1])
    a16 = a.astype(bf16)
    bT16 = b.T.astype(bf16)
    w2 = e_w2_ref[1].astype(bf16)
    wd16 = w_d_ref[1].astype(bf16)
    dist16 = dist.astype(bf16)
    F = jnp.zeros((P, P), bf16)
    for m in range(16):
        u = (a16[:, m:m + 1] + bT16[m:m + 1, :] + ea1_ref[m]
             + dist16 * wd16[0:1, m:m + 1])
        F = F + jnp.maximum(u, jnp.bfloat16(0.0)) * w2[0:1, m:m + 1]
    s = jnp.sum(mask * F.astype(jnp.float32), axis=1, keepdims=True)
    e_mean = s * (1.0 / K) + e_b2_ref[1]

    g = jnp.maximum(e_mean * c_w1_ref[1] + c_b1_ref[1], 0.0)
    dpos = _dotf(g, c_w2_ref[1]) + c_b2_ref[1]
    out2_ref[0] = posc + dpos


def _full(shape):
    rank = len(shape)
    return pl.BlockSpec(shape, lambda *_: (0,) * rank)


@jax.jit
def kernel(x, pos, edge_attr, params):
    f32 = jnp.float32

    CB = 4096
    n_cb = (P * H) // CB
    h0 = pl.pallas_call(
        _embed_kernel,
        grid=(n_cb,),
        in_specs=[
            pl.BlockSpec((N, IN_DIM), lambda i: (0, 0)),
            pl.BlockSpec((IN_DIM, CB), lambda i: (0, i)),
            pl.BlockSpec((1, CB), lambda i: (0, i)),
        ],
        out_specs=pl.BlockSpec((N, CB), lambda i: (0, i)),
        out_shape=jax.ShapeDtypeStruct((N, P * H), f32),
        compiler_params=pltpu.CompilerParams(
            dimension_semantics=("arbitrary",)),
        interpret=_INTERPRET,
    )(x, params['z_W'], params['z_b'].reshape(1, P * H))
    h0 = h0.reshape(N, P, H)

    posc = pos.astype(f32)
    posr = posc.T
    eaT = edge_attr.T

    lp = params['layers']

    def stack(name):
        return jnp.stack([lp[l][name] for l in range(LAYERS)])

    e_W1 = stack('e_W1')
    w1a = e_W1[:, :H, :]
    w1b = e_W1[:, H:2 * H, :]
    w_ea = e_W1[:, 2 * H:2 * H + 1, :]
    w_d = e_W1[:, 2 * H + 1:2 * H + 2, :]
    e_b1 = stack('e_b1').reshape(LAYERS, 1, 16)
    e_w2 = stack('e_W2').reshape(LAYERS, 1, 16)
    e_b2 = stack('e_b2').reshape(LAYERS, 1, 1)
    n_W1 = stack('n_W1')
    n_w1h = n_W1[:, :H, :]
    n_w1e = n_W1[:, H:H + 1, :]
    n_b1 = stack('n_b1').reshape(LAYERS, 1, 16)
    n_w2 = stack('n_W2')
    n_b2 = stack('n_b2').reshape(LAYERS, 1, H)
    c_w1 = stack('c_W1')
    c_b1 = stack('c_b1').reshape(LAYERS, 1, 16)
    c_w2 = stack('c_W2')
    c_b2 = stack('c_b2').reshape(LAYERS, 1, 3)

    mask0, c0, ea1 = pl.pallas_call(
        _knn0_kernel,
        in_specs=[_full((P, 3)), _full((3, P)), _full((P, P)),
                  _full((LAYERS, 1, 16)), _full((LAYERS, 1, 16))],
        out_specs=[_full((P, P)), _full((16, P, P)), _full((16, P, P))],
        out_shape=[jax.ShapeDtypeStruct((P, P), f32),
                   jax.ShapeDtypeStruct((16, P, P), jnp.bfloat16),
                   jax.ShapeDtypeStruct((16, P, P), jnp.bfloat16)],
        interpret=_INTERPRET,
    )(posc, posr, eaT, w_ea, w_d)

    in_specs = [
        pl.BlockSpec((1, P, H), lambda n: (n, 0, 0)),
        _full((P, 3)), _full((P, P)),
        _full((16, P, P)), _full((16, P, P)),
        _full((H, H)), _full((1, H)), _full((H, H)), _full((1, H)),
        _full((H, H)), _full((1, H)), _full((H, H)), _full((1, H)),
        _full((H, H)), _full((1, H)),
        _full((LAYERS, H, 16)), _full((LAYERS, H, 16)),
        _full((LAYERS, 1, 16)), _full((LAYERS, 1, 16)),
        _full((LAYERS, 1, 16)), _full((LAYERS, 1, 16)),
        _full((LAYERS, 1, 1)),
        _full((LAYERS, H, 16)), _full((LAYERS, 1, 16)),
        _full((LAYERS, 1, 16)), _full((LAYERS, 16, H)),
        _full((LAYERS, 1, H)),
        _full((LAYERS, 1, 16)), _full((LAYERS, 1, 16)),
        _full((LAYERS, 16, 3)), _full((LAYERS, 1, 3)),
    ]
    out_spec = pl.BlockSpec((1, P, 3), lambda n: (n, 0, 0))
    c1, c2 = pl.pallas_call(
        _graph_kernel,
        grid=(N,),
        in_specs=in_specs,
        out_specs=[out_spec, out_spec],
        out_shape=[jax.ShapeDtypeStruct((N, P, 3), f32),
                   jax.ShapeDtypeStruct((N, P, 3), f32)],
        compiler_params=pltpu.CompilerParams(
            dimension_semantics=("parallel",)),
        interpret=_INTERPRET,
    )(h0, posc, mask0, c0, ea1,
      params['Wq'], params['bq'].reshape(1, H),
      params['Wk'], params['bk'].reshape(1, H),
      params['Wv'], params['bv'].reshape(1, H),
      params['Wo'], params['bo'].reshape(1, H),
      params['lin_W'], params['lin_b'].reshape(1, H),
      w1a, w1b, w_ea, w_d, e_b1, e_w2, e_b2,
      n_w1h, n_w1e, n_b1, n_w2, n_b2,
      c_w1, c_b1, c_w2, c_b2)

    return (c1.reshape(N * P, 3), c2.reshape(N * P, 3))


# speedup vs baseline: 1.9056x; 1.0247x over previous
import math

import jax
import jax.numpy as jnp
from jax.experimental import pallas as pl
from jax.experimental.pallas import tpu as pltpu

N = 64
IN_DIM = 128
P = 256
H = 128
HEADS = 4
DH = H // HEADS
K = 32
LAYERS = 2
GPB = 2

_INTERPRET = False


def _dotf(a, b):
    return jnp.dot(a, b, preferred_element_type=jnp.float32)


def _dot_nt(a, b):
    return jax.lax.dot_general(a, b, (((1,), (1,)), ((), ())),
                               preferred_element_type=jnp.float32)


def _pairwise_d2(posc, posr):
    d2 = jnp.zeros((P, P), jnp.float32)
    for c in range(3):
        diff = posc[:, c:c + 1] - posr[c:c + 1, :]
        d2 = d2 + diff * diff
    ri = jax.lax.broadcasted_iota(jnp.int32, (P, P), 0)
    ci = jax.lax.broadcasted_iota(jnp.int32, (P, P), 1)
    return jnp.where(ri == ci, 1e10, d2)


def _knn_mask(d2):
    work = d2
    for _ in range(K):
        m = jnp.min(work, axis=1, keepdims=True)
        work = jnp.where(work <= m, jnp.float32(3e38), work)
    return jnp.where(work >= 1e38, 1.0, 0.0)


def _embed_kernel(x_ref, w_ref, b_ref, o_ref):
    o_ref[...] = _dotf(x_ref[...], w_ref[...]) + b_ref[...]


def _knn0_kernel(posc_ref, posr_ref, eaT_ref, w_ea_ref, w_d_ref,
                 mask_ref, c0_ref, ea1_ref):
    d2 = _pairwise_d2(posc_ref[...], posr_ref[...])
    dist0 = jnp.sqrt(d2)
    mask_ref[...] = _knn_mask(d2)
    eaT = eaT_ref[...]
    for m in range(16):
        c0 = (eaT * w_ea_ref[0, 0:1, m:m + 1]
              + dist0 * w_d_ref[0, 0:1, m:m + 1])
        c0_ref[m] = c0.astype(jnp.bfloat16)
        ea1_ref[m] = (eaT * w_ea_ref[1, 0:1, m:m + 1]).astype(jnp.bfloat16)


def _graph_kernel(h0_ref, posc_ref, mask0_ref, c0_ref, ea1_ref,
                  wq_ref, bq_ref, wk_ref, bk_ref, wv_ref, bv_ref,
                  wo_ref, bo_ref, lin_w_ref, lin_b_ref,
                  w1a_ref, w1b_ref, w_ea_ref, w_d_ref, e_b1_ref,
                  e_w2_ref, e_b2_ref,
                  n_w1h_ref, n_w1e_ref, n_b1_ref, n_w2_ref, n_b2_ref,
                  c_w1_ref, c_b1_ref, c_w2_ref, c_b2_ref,
                  out1_ref, out2_ref):
    for gidx in range(GPB):
        _one_graph(h0_ref, posc_ref, mask0_ref, c0_ref, ea1_ref,
                   wq_ref, bq_ref, wk_ref, bk_ref, wv_ref, bv_ref,
                   wo_ref, bo_ref, lin_w_ref, lin_b_ref,
                   w1a_ref, w1b_ref, w_ea_ref, w_d_ref, e_b1_ref,
                   e_w2_ref, e_b2_ref,
                   n_w1h_ref, n_w1e_ref, n_b1_ref, n_w2_ref, n_b2_ref,
                   c_w1_ref, c_b1_ref, c_w2_ref, c_b2_ref,
                   out1_ref, out2_ref, gidx)


def _one_graph(h0_ref, posc_ref, mask0_ref, c0_ref, ea1_ref,
               wq_ref, bq_ref, wk_ref, bk_ref, wv_ref, bv_ref,
               wo_ref, bo_ref, lin_w_ref, lin_b_ref,
               w1a_ref, w1b_ref, w_ea_ref, w_d_ref, e_b1_ref,
               e_w2_ref, e_b2_ref,
               n_w1h_ref, n_w1e_ref, n_b1_ref, n_w2_ref, n_b2_ref,
               c_w1_ref, c_b1_ref, c_w2_ref, c_b2_ref,
               out1_ref, out2_ref, gidx):
    h = h0_ref[gidx]

    q = _dotf(h, wq_ref[...]) + bq_ref[...]
    k = _dotf(h, wk_ref[...]) + bk_ref[...]
    v = _dotf(h, wv_ref[...]) + bv_ref[...]
    scale = 1.0 / math.sqrt(DH)
    heads = []
    for hd in range(HEADS):
        sl = slice(hd * DH, (hd + 1) * DH)
        s = _dot_nt(q[:, sl], k[:, sl]) * scale
        e = jnp.exp(s)
        pattn = e / jnp.sum(e, axis=1, keepdims=True)
        heads.append(_dotf(pattn, v[:, sl]))
    o = jnp.concatenate(heads, axis=1)
    o = _dotf(o, wo_ref[...]) + bo_ref[...]
    h = _dotf(o, lin_w_ref[...]) + lin_b_ref[...]

    bf16 = jnp.bfloat16
    a = _dotf(h, w1a_ref[0]) + e_b1_ref[0]
    b = _dotf(h, w1b_ref[0])
    a16 = a.astype(bf16)
    bT16 = b.T.astype(bf16)
    w2 = e_w2_ref[0].astype(bf16)
    F = jnp.zeros((P, P), bf16)
    for m in range(16):
        u = a16[:, m:m + 1] + bT16[m:m + 1, :] + c0_ref[m]
        F = F + jnp.maximum(u, jnp.bfloat16(0.0)) * w2[0:1, m:m + 1]
    s = jnp.sum(mask0_ref[...] * F.astype(jnp.float32), axis=1,
                keepdims=True)
    e_mean = s * (1.0 / K) + e_b2_ref[0]

    pre = _dotf(h, n_w1h_ref[0]) + e_mean * n_w1e_ref[0] + n_b1_ref[0]
    h = _dotf(jnp.maximum(pre, 0.0), n_w2_ref[0]) + n_b2_ref[0]

    g = jnp.maximum(e_mean * c_w1_ref[0] + c_b1_ref[0], 0.0)
    dpos = _dotf(g, c_w2_ref[0]) + c_b2_ref[0]
    posc = posc_ref[...] + dpos
    out1_ref[gidx] = posc

    posr = posc.T
    d2 = _pairwise_d2(posc, posr)
    dist = jnp.sqrt(d2)
    mask = _knn_mask(d2)

    a = _dotf(h, w1a_ref[1]) + e_b1_ref[1]
    b = _dotf(h, w1b_ref[1])
    a16 = a.astype(bf16)
    bT16 = b.T.astype(bf16)
    w2 = e_w2_ref[1].astype(bf16)
    wd16 = w_d_ref[1].astype(bf16)
    dist16 = dist.astype(bf16)
    F = jnp.zeros((P, P), bf16)
    for m in range(16):
        u = (a16[:, m:m + 1] + bT16[m:m + 1, :] + ea1_ref[m]
             + dist16 * wd16[0:1, m:m + 1])
        F = F + jnp.maximum(u, jnp.bfloat16(0.0)) * w2[0:1, m:m + 1]
    s = jnp.sum(mask * F.astype(jnp.float32), axis=1, keepdims=True)
    e_mean = s * (1.0 / K) + e_b2_ref[1]

    g = jnp.maximum(e_mean * c_w1_ref[1] + c_b1_ref[1], 0.0)
    dpos = _dotf(g, c_w2_ref[1]) + c_b2_ref[1]
    out2_ref[gidx] = posc + dpos


def _full(shape):
    rank = len(shape)
    return pl.BlockSpec(shape, lambda *_: (0,) * rank)


@jax.jit
def kernel(x, pos, edge_attr, params):
    f32 = jnp.float32

    CB = 4096
    n_cb = (P * H) // CB
    h0 = pl.pallas_call(
        _embed_kernel,
        grid=(n_cb,),
        in_specs=[
            pl.BlockSpec((N, IN_DIM), lambda i: (0, 0)),
            pl.BlockSpec((IN_DIM, CB), lambda i: (0, i)),
            pl.BlockSpec((1, CB), lambda i: (0, i)),
        ],
        out_specs=pl.BlockSpec((N, CB), lambda i: (0, i)),
        out_shape=jax.ShapeDtypeStruct((N, P * H), f32),
        compiler_params=pltpu.CompilerParams(
            dimension_semantics=("arbitrary",)),
        interpret=_INTERPRET,
    )(x, params['z_W'], params['z_b'].reshape(1, P * H))
    h0 = h0.reshape(N, P, H)

    posc = pos.astype(f32)
    posr = posc.T
    eaT = edge_attr.T

    lp = params['layers']

    def stack(name):
        return jnp.stack([lp[l][name] for l in range(LAYERS)])

    e_W1 = stack('e_W1')
    w1a = e_W1[:, :H, :]
    w1b = e_W1[:, H:2 * H, :]
    w_ea = e_W1[:, 2 * H:2 * H + 1, :]
    w_d = e_W1[:, 2 * H + 1:2 * H + 2, :]
    e_b1 = stack('e_b1').reshape(LAYERS, 1, 16)
    e_w2 = stack('e_W2').reshape(LAYERS, 1, 16)
    e_b2 = stack('e_b2').reshape(LAYERS, 1, 1)
    n_W1 = stack('n_W1')
    n_w1h = n_W1[:, :H, :]
    n_w1e = n_W1[:, H:H + 1, :]
    n_b1 = stack('n_b1').reshape(LAYERS, 1, 16)
    n_w2 = stack('n_W2')
    n_b2 = stack('n_b2').reshape(LAYERS, 1, H)
    c_w1 = stack('c_W1')
    c_b1 = stack('c_b1').reshape(LAYERS, 1, 16)
    c_w2 = stack('c_W2')
    c_b2 = stack('c_b2').reshape(LAYERS, 1, 3)

    mask0, c0, ea1 = pl.pallas_call(
        _knn0_kernel,
        in_specs=[_full((P, 3)), _full((3, P)), _full((P, P)),
                  _full((LAYERS, 1, 16)), _full((LAYERS, 1, 16))],
        out_specs=[_full((P, P)), _full((16, P, P)), _full((16, P, P))],
        out_shape=[jax.ShapeDtypeStruct((P, P), f32),
                   jax.ShapeDtypeStruct((16, P, P), jnp.bfloat16),
                   jax.ShapeDtypeStruct((16, P, P), jnp.bfloat16)],
        interpret=_INTERPRET,
    )(posc, posr, eaT, w_ea, w_d)

    in_specs = [
        pl.BlockSpec((GPB, P, H), lambda n: (n, 0, 0)),
        _full((P, 3)), _full((P, P)),
        _full((16, P, P)), _full((16, P, P)),
        _full((H, H)), _full((1, H)), _full((H, H)), _full((1, H)),
        _full((H, H)), _full((1, H)), _full((H, H)), _full((1, H)),
        _full((H, H)), _full((1, H)),
        _full((LAYERS, H, 16)), _full((LAYERS, H, 16)),
        _full((LAYERS, 1, 16)), _full((LAYERS, 1, 16)),
        _full((LAYERS, 1, 16)), _full((LAYERS, 1, 16)),
        _full((LAYERS, 1, 1)),
        _full((LAYERS, H, 16)), _full((LAYERS, 1, 16)),
        _full((LAYERS, 1, 16)), _full((LAYERS, 16, H)),
        _full((LAYERS, 1, H)),
        _full((LAYERS, 1, 16)), _full((LAYERS, 1, 16)),
        _full((LAYERS, 16, 3)), _full((LAYERS, 1, 3)),
    ]
    out_spec = pl.BlockSpec((GPB, P, 3), lambda n: (n, 0, 0))
    c1, c2 = pl.pallas_call(
        _graph_kernel,
        grid=(N // GPB,),
        in_specs=in_specs,
        out_specs=[out_spec, out_spec],
        out_shape=[jax.ShapeDtypeStruct((N, P, 3), f32),
                   jax.ShapeDtypeStruct((N, P, 3), f32)],
        compiler_params=pltpu.CompilerParams(
            dimension_semantics=("parallel",)),
        interpret=_INTERPRET,
    )(h0, posc, mask0, c0, ea1,
      params['Wq'], params['bq'].reshape(1, H),
      params['Wk'], params['bk'].reshape(1, H),
      params['Wv'], params['bv'].reshape(1, H),
      params['Wo'], params['bo'].reshape(1, H),
      params['lin_W'], params['lin_b'].reshape(1, H),
      w1a, w1b, w_ea, w_d, e_b1, e_w2, e_b2,
      n_w1h, n_w1e, n_b1, n_w2, n_b2,
      c_w1, c_b1, c_w2, c_b2)

    return (c1.reshape(N * P, 3), c2.reshape(N * P, 3))
